# baseline (speedup 1.0000x reference)
import functools

import jax
import jax.numpy as jnp
from jax import lax
from jax.experimental import pallas as pl
from jax.experimental.pallas import tpu as pltpu
from jax.experimental.pallas import tpu_sc as plsc

N = 50000
E = 800000
H = 64
G = 512
NPAD = 51200
KB = 5
GROUPS = 80
EPT = GROUPS * KB * 128
EPAD = 16 * EPT
GPAD = GROUPS + 1
RB = 2048
NB = NPAD // RB
HHALF = 32
ACC_ROWS = 50048
ZR = 3128

_mesh = plsc.VectorSubcoreMesh(core_axis_name="c", subcore_axis_name="s")


def _agg_body(h2, icomb, out,
              i0, i1, rows, acc,
              sg, si0, si1):
    c = lax.axis_index("c")
    w = lax.axis_index("s")
    ibufs = (i0, i1)
    isems = (si0, si1)

    def zrow(i, carry):
        rows[i, pl.ds(0, 16)] = jnp.zeros((16,), jnp.float32)
        rows[i, pl.ds(16, 16)] = jnp.zeros((16,), jnp.float32)
        return carry

    lax.fori_loop(0, 128, zrow, 0)
    z0 = w * ZR
    for q in range(ZR // 128):
        pltpu.sync_copy(rows.at[pl.ds(0, 128)], acc.at[pl.ds(z0 + q * 128, 128)])
    pltpu.sync_copy(rows.at[pl.ds(0, ZR % 128)],
                    acc.at[pl.ds(z0 + (ZR // 128) * 128, ZR % 128)])
    plsc.subcore_barrier()

    pltpu.async_copy(icomb.at[c, w, 0], ibufs[0], isems[0])

    def body2(k, carry):
        for b in range(2):
            g = 2 * k + b
            bn = 1 - b
            pltpu.make_async_copy(icomb.at[c, w, g], ibufs[b], isems[b]).wait()
            cps = [
                pltpu.async_copy(
                    h2.at[ibufs[b].at[0, j]],
                    rows.at[pl.ds(j * 128, 128)], sg)
                for j in range(KB)
            ]
            pltpu.async_copy(icomb.at[c, w, g + 1], ibufs[bn], isems[bn])
            for cp in cps:
                cp.wait()
            for j in range(KB):
                pltpu.sync_copy(
                    rows.at[pl.ds(j * 128, 128)],
                    acc.at[ibufs[b].at[1, j]], add=True)
        return carry

    lax.fori_loop(0, GROUPS // 2, body2, 0)
    pltpu.make_async_copy(icomb.at[c, w, 0], ibufs[0], isems[0]).wait()

    plsc.subcore_barrier()
    pltpu.sync_copy(
        acc.at[pl.ds(w * 3125, 3125)],
        out.at[pl.ds(w * 3125, 3125), pl.ds(c * HHALF, HHALF)],
    )


_agg_call = functools.partial(
    pl.kernel,
    out_type=jax.ShapeDtypeStruct((NPAD, H), jnp.float32),
    mesh=_mesh,
    compiler_params=pltpu.CompilerParams(use_tc_tiling_on_sc=False),
    scratch_types=[
        pltpu.VMEM((2, KB, 128), jnp.int32),
        pltpu.VMEM((2, KB, 128), jnp.int32),
        pltpu.VMEM((KB * 128, HHALF), jnp.float32),
        pltpu.VMEM_SHARED((ACC_ROWS, HHALF), jnp.float32),
        pltpu.SemaphoreType.DMA,
        pltpu.SemaphoreType.DMA,
        pltpu.SemaphoreType.DMA,
    ],
)(_agg_body)


def _pool_body(h, batch_r, zrs, ones_h, psums, pcnts, bidx, rowsv, onesv,
               accs, accc, sem):
    c = lax.axis_index("c")
    w = lax.axis_index("s")

    @pl.when(w == 0)
    def _():
        pltpu.sync_copy(zrs, accs)

    @pl.when(w == 1)
    def _():
        pltpu.sync_copy(zrs, accc)

    pltpu.sync_copy(ones_h, onesv)
    plsc.subcore_barrier()

    def body(g, carry):
        row0 = w * 3200 + g * 128
        pltpu.sync_copy(batch_r.at[w, g], bidx)
        pltpu.async_copy(
            h.at[pl.ds(row0, 128), pl.ds(c * HHALF, HHALF)], rowsv, sem
        ).wait()
        pltpu.sync_copy(rowsv, accs.at[bidx.at[0]], add=True)
        pltpu.sync_copy(onesv, accc.at[bidx.at[0]], add=True)
        return carry

    lax.fori_loop(0, 25, body, 0)
    plsc.subcore_barrier()

    @pl.when(w == 0)
    def _():
        pltpu.sync_copy(accs.at[pl.ds(0, G)], psums.at[c])

    @pl.when(w == 1)
    def _():
        pltpu.sync_copy(accc.at[pl.ds(0, G)], pcnts.at[c])


_pool_call = functools.partial(
    pl.kernel,
    out_type=(
        jax.ShapeDtypeStruct((2, G, HHALF), jnp.float32),
        jax.ShapeDtypeStruct((2, G, HHALF), jnp.float32),
    ),
    mesh=_mesh,
    compiler_params=pltpu.CompilerParams(use_tc_tiling_on_sc=False),
    scratch_types=[
        pltpu.VMEM((1, 128), jnp.int32),
        pltpu.VMEM((128, HHALF), jnp.float32),
        pltpu.VMEM((128, HHALF), jnp.float32),
        pltpu.VMEM_SHARED((G + 8, HHALF), jnp.float32),
        pltpu.VMEM_SHARED((G + 8, HHALF), jnp.float32),
        pltpu.SemaphoreType.DMA,
    ],
)(_pool_body)


def _proj_body(xr, wr, br, orf):
    orf[...] = jnp.dot(xr[...], wr[...], preferred_element_type=jnp.float32) + br[...]


def _proj(x_pad, wp, bp):
    return pl.pallas_call(
        _proj_body,
        grid=(NB,),
        in_specs=[
            pl.BlockSpec((RB, 128), lambda i: (i, 0)),
            pl.BlockSpec((128, H), lambda i: (0, 0)),
            pl.BlockSpec((1, H), lambda i: (0, 0)),
        ],
        out_specs=pl.BlockSpec((RB, H), lambda i: (i, 0)),
        out_shape=jax.ShapeDtypeStruct((NPAD, H), jnp.float32),
    )(x_pad, wp, bp)


def _mlp_body(hr, ar, er, w1, b1, w2, b2, tr, sr, ssum, ssq):
    i = pl.program_id(0)
    m = (1.0 + er[0, 0]) * hr[...] + ar[...]
    z = jnp.maximum(jnp.dot(m, w1[...], preferred_element_type=jnp.float32) + b1[...], 0.0)
    t = jnp.dot(z, w2[...], preferred_element_type=jnp.float32) + b2[...]
    tr[...] = t
    rows = i * RB + lax.broadcasted_iota(jnp.int32, (RB, H), 0)
    tm = jnp.where(rows < N, t, 0.0)

    @pl.when(i == 0)
    def _():
        ssum[...] = jnp.zeros_like(ssum)
        ssq[...] = jnp.zeros_like(ssq)

    ssum[...] += jnp.sum(tm, axis=0, keepdims=True)
    ssq[...] += jnp.sum(tm * tm, axis=0, keepdims=True)

    @pl.when(i == pl.num_programs(0) - 1)
    def _():
        sr[0:1, :] = ssum[...]
        sr[1:2, :] = ssq[...]


def _mlp(h, agg, eps, w1, b1, w2, b2):
    return pl.pallas_call(
        _mlp_body,
        grid=(NB,),
        in_specs=[
            pl.BlockSpec((RB, H), lambda i: (i, 0)),
            pl.BlockSpec((RB, H), lambda i: (i, 0)),
            pl.BlockSpec((1, 1), lambda i: (0, 0)),
            pl.BlockSpec((H, H), lambda i: (0, 0)),
            pl.BlockSpec((1, H), lambda i: (0, 0)),
            pl.BlockSpec((H, H), lambda i: (0, 0)),
            pl.BlockSpec((1, H), lambda i: (0, 0)),
        ],
        out_specs=[
            pl.BlockSpec((RB, H), lambda i: (i, 0)),
            pl.BlockSpec((2, H), lambda i: (0, 0)),
        ],
        out_shape=[
            jax.ShapeDtypeStruct((NPAD, H), jnp.float32),
            jax.ShapeDtypeStruct((2, H), jnp.float32),
        ],
        scratch_shapes=[
            pltpu.VMEM((1, H), jnp.float32),
            pltpu.VMEM((1, H), jnp.float32),
        ],
    )(h, agg, eps, w1, b1, w2, b2)


def _bn_body(tr, sr, gr, br, orf):
    mean = sr[0:1, :] * (1.0 / N)
    var = sr[1:2, :] * (1.0 / N) - mean * mean
    inv = lax.rsqrt(var + 1e-5)
    scale = gr[...] * inv
    shift = br[...] - mean * scale
    orf[...] = jnp.maximum(tr[...] * scale + shift, 0.0)


def _bn(t, stats, gamma, beta):
    return pl.pallas_call(
        _bn_body,
        grid=(NB,),
        in_specs=[
            pl.BlockSpec((RB, H), lambda i: (i, 0)),
            pl.BlockSpec((2, H), lambda i: (0, 0)),
            pl.BlockSpec((1, H), lambda i: (0, 0)),
            pl.BlockSpec((1, H), lambda i: (0, 0)),
        ],
        out_specs=pl.BlockSpec((RB, H), lambda i: (i, 0)),
        out_shape=jax.ShapeDtypeStruct((NPAD, H), jnp.float32),
    )(t, stats, gamma, beta)


def _readout_body(ps, cs, w1, b1, w2, b2, orf):
    p = jnp.concatenate([ps[0], ps[1]], axis=1)
    cnt = cs[0, :, 0:1]
    pooled = p / jnp.maximum(cnt, 1.0)
    z = jnp.maximum(jnp.dot(pooled, w1[...], preferred_element_type=jnp.float32) + b1[...], 0.0)
    orf[...] = jnp.dot(z, w2[...], preferred_element_type=jnp.float32) + b2[...]


def _readout(psums, pcnts, w1, b1, w2, b2):
    return pl.pallas_call(
        _readout_body,
        out_shape=jax.ShapeDtypeStruct((G, 1), jnp.float32),
    )(psums, pcnts, w1, b1, w2, b2)


def kernel(x, edge_index, batch, params):
    din = x.shape[1]
    x_pad = jnp.zeros((NPAD, 128), jnp.float32).at[:N, :din].set(x)
    wp = jnp.zeros((128, H), jnp.float32).at[:din].set(params["node_proj"]["W"])
    bp = params["node_proj"]["b"].reshape(1, H)

    src = edge_index[0]
    dst = edge_index[1]
    srcp = jnp.concatenate([src, jnp.zeros((EPAD - E,), jnp.int32)])
    dstp = jnp.concatenate([dst, jnp.full((EPAD - E,), N, jnp.int32)])
    sidx_r = jnp.stack([2 * srcp, 2 * srcp + 1]).reshape(2, 16, GROUPS, 1, KB, 128)
    didx_r = jnp.broadcast_to(
        dstp.reshape(1, 16, GROUPS, 1, KB, 128), (2, 16, GROUPS, 1, KB, 128)
    )
    icomb = jnp.concatenate([sidx_r, didx_r], axis=3)
    icomb = jnp.concatenate(
        [icomb, jnp.zeros((2, 16, GPAD - GROUPS, 2, KB, 128), jnp.int32)], axis=2
    )
    batch_r = jnp.concatenate(
        [batch, jnp.full((NPAD - N,), G, jnp.int32)]
    ).reshape(16, 25, 1, 128)

    zpool = jnp.zeros((G + 8, HHALF), jnp.float32)
    ones = jnp.ones((128, HHALF), jnp.float32)

    h = _proj(x_pad, wp, bp)
    for blk in params["blocks"]:
        h2 = h.reshape(2 * NPAD, HHALF)
        agg = _agg_call(h2, icomb)
        t, stats = _mlp(
            h, agg, blk["eps"].reshape(1, 1),
            blk["W1"], blk["b1"].reshape(1, H),
            blk["W2"], blk["b2"].reshape(1, H),
        )
        h = _bn(t, stats, blk["gamma"].reshape(1, H), blk["beta"].reshape(1, H))

    psums, pcnts = _pool_call(h, batch_r, zpool, ones)
    r = params["readout"]
    return _readout(
        psums, pcnts, r["W1"], r["b1"].reshape(1, H), r["W2"], r["b2"].reshape(1, 1)
    )


# speedup vs baseline: 1.6770x; 1.6770x over previous
import functools

import jax
import jax.numpy as jnp
from jax import lax
from jax.experimental import pallas as pl
from jax.experimental.pallas import tpu as pltpu
from jax.experimental.pallas import tpu_sc as plsc

N = 50000
E = 800000
H = 64
G = 512
NPAD = 51200
KB = 4
GROUPS = 98
EPT = GROUPS * KB * 128
EPAD = 16 * EPT
GPAD = GROUPS + 1
RB = 2048
NB = NPAD // RB
HHALF = 32
ACC_ROWS = 50048
ZR = 3128

_mesh = plsc.VectorSubcoreMesh(core_axis_name="c", subcore_axis_name="s")


def _agg_body(h2, icomb, out,
              i0, i1, rows, acc,
              sg, si0, si1):
    c = lax.axis_index("c")
    w = lax.axis_index("s")
    ibufs = (i0, i1)
    isems = (si0, si1)

    def zrow(i, carry):
        rows[i, pl.ds(0, 16)] = jnp.zeros((16,), jnp.float32)
        rows[i, pl.ds(16, 16)] = jnp.zeros((16,), jnp.float32)
        return carry

    lax.fori_loop(0, 128, zrow, 0)
    z0 = w * ZR
    for q in range(ZR // 128):
        pltpu.sync_copy(rows.at[pl.ds(0, 128)], acc.at[pl.ds(z0 + q * 128, 128)])
    pltpu.sync_copy(rows.at[pl.ds(0, ZR % 128)],
                    acc.at[pl.ds(z0 + (ZR // 128) * 128, ZR % 128)])
    plsc.subcore_barrier()

    pltpu.async_copy(icomb.at[c, w, 0], ibufs[0], isems[0])

    def body2(k, carry):
        for b in range(2):
            g = 2 * k + b
            bn = 1 - b
            pltpu.make_async_copy(icomb.at[c, w, g], ibufs[b], isems[b]).wait()
            cps = [
                pltpu.async_copy(
                    h2.at[ibufs[b].at[0, j]],
                    rows.at[pl.ds(j * 128, 128)], sg)
                for j in range(KB)
            ]
            pltpu.async_copy(icomb.at[c, w, g + 1], ibufs[bn], isems[bn])
            for cp in cps:
                cp.wait()
            for j in range(KB):
                pltpu.sync_copy(
                    rows.at[pl.ds(j * 128, 128)],
                    acc.at[ibufs[b].at[1, j]], add=True)
        return carry

    lax.fori_loop(0, GROUPS // 2, body2, 0)
    pltpu.make_async_copy(icomb.at[c, w, 0], ibufs[0], isems[0]).wait()

    plsc.subcore_barrier()
    pltpu.sync_copy(
        acc.at[pl.ds(w * 3125, 3125)],
        out.at[pl.ds(w * 3125, 3125), pl.ds(c * HHALF, HHALF)],
    )


_agg_call = functools.partial(
    pl.kernel,
    out_type=jax.ShapeDtypeStruct((NPAD, H), jnp.float32),
    mesh=_mesh,
    compiler_params=pltpu.CompilerParams(use_tc_tiling_on_sc=False),
    scratch_types=[
        pltpu.VMEM((2, KB, 128), jnp.int32),
        pltpu.VMEM((2, KB, 128), jnp.int32),
        pltpu.VMEM((KB * 128, HHALF), jnp.float32),
        pltpu.VMEM_SHARED((ACC_ROWS, HHALF), jnp.float32),
        pltpu.SemaphoreType.DMA,
        pltpu.SemaphoreType.DMA,
        pltpu.SemaphoreType.DMA,
    ],
)(_agg_body)


def _pool_body(h, batch_r, zrs, ones_h, psums, pcnts, bidx, r0, r1, onesv,
               accs, accc, sr0, sr1, sem):
    c = lax.axis_index("c")
    w = lax.axis_index("s")
    rbufs = (r0, r1)
    rsems = (sr0, sr1)

    @pl.when(w == 0)
    def _():
        pltpu.sync_copy(zrs, accs)

    @pl.when(w == 1)
    def _():
        pltpu.sync_copy(zrs, accc)

    pltpu.sync_copy(ones_h, onesv)
    pltpu.sync_copy(batch_r.at[w], bidx)
    plsc.subcore_barrier()

    def fire(g, b):
        pltpu.async_copy(
            h.at[pl.ds(w * 3200 + g * 128, 128), pl.ds(c * HHALF, HHALF)],
            rbufs[b], rsems[b])

    def wait(g, b):
        pltpu.make_async_copy(
            h.at[pl.ds(w * 3200 + g * 128, 128), pl.ds(c * HHALF, HHALF)],
            rbufs[b], rsems[b]).wait()

    fire(0, 0)

    def body2(k, carry):
        for b in range(2):
            g = 2 * k + b
            wait(g, b)
            fire(g + 1, 1 - b)
            pltpu.sync_copy(rbufs[b], accs.at[bidx.at[g]], add=True)
            pltpu.sync_copy(onesv, accc.at[bidx.at[g]], add=True)
        return carry

    lax.fori_loop(0, 12, body2, 0)
    wait(24, 0)
    pltpu.sync_copy(rbufs[0], accs.at[bidx.at[24]], add=True)
    pltpu.sync_copy(onesv, accc.at[bidx.at[24]], add=True)
    plsc.subcore_barrier()

    @pl.when(w == 0)
    def _():
        pltpu.sync_copy(accs.at[pl.ds(0, G)], psums.at[c])

    @pl.when(w == 1)
    def _():
        pltpu.sync_copy(accc.at[pl.ds(0, G)], pcnts.at[c])


_pool_call = functools.partial(
    pl.kernel,
    out_type=(
        jax.ShapeDtypeStruct((2, G, HHALF), jnp.float32),
        jax.ShapeDtypeStruct((2, G, HHALF), jnp.float32),
    ),
    mesh=_mesh,
    compiler_params=pltpu.CompilerParams(use_tc_tiling_on_sc=False),
    scratch_types=[
        pltpu.VMEM((25, 128), jnp.int32),
        pltpu.VMEM((128, HHALF), jnp.float32),
        pltpu.VMEM((128, HHALF), jnp.float32),
        pltpu.VMEM((128, HHALF), jnp.float32),
        pltpu.VMEM_SHARED((G + 8, HHALF), jnp.float32),
        pltpu.VMEM_SHARED((G + 8, HHALF), jnp.float32),
        pltpu.SemaphoreType.DMA,
        pltpu.SemaphoreType.DMA,
        pltpu.SemaphoreType.DMA,
    ],
)(_pool_body)


def _proj_body(xr, wr, br, orf):
    orf[...] = jnp.dot(xr[...], wr[...], preferred_element_type=jnp.float32) + br[...]


def _proj(x, wp, bp):
    din = x.shape[1]
    return pl.pallas_call(
        _proj_body,
        grid=(25,),
        in_specs=[
            pl.BlockSpec((2000, din), lambda i: (i, 0)),
            pl.BlockSpec((din, H), lambda i: (0, 0)),
            pl.BlockSpec((1, H), lambda i: (0, 0)),
        ],
        out_specs=pl.BlockSpec((2000, H), lambda i: (i, 0)),
        out_shape=jax.ShapeDtypeStruct((NPAD, H), jnp.float32),
    )(x, wp, bp)


def _mlp_body(hr, ar, er, w1, b1, w2, b2, tr, sr, ssum, ssq):
    i = pl.program_id(0)
    m = (1.0 + er[0, 0]) * hr[...] + ar[...]
    z = jnp.maximum(jnp.dot(m, w1[...], preferred_element_type=jnp.float32) + b1[...], 0.0)
    t = jnp.dot(z, w2[...], preferred_element_type=jnp.float32) + b2[...]
    tr[...] = t
    rows = i * RB + lax.broadcasted_iota(jnp.int32, (RB, H), 0)
    tm = jnp.where(rows < N, t, 0.0)

    @pl.when(i == 0)
    def _():
        ssum[...] = jnp.zeros_like(ssum)
        ssq[...] = jnp.zeros_like(ssq)

    ssum[...] += jnp.sum(tm, axis=0, keepdims=True)
    ssq[...] += jnp.sum(tm * tm, axis=0, keepdims=True)

    @pl.when(i == pl.num_programs(0) - 1)
    def _():
        sr[0:1, :] = ssum[...]
        sr[1:2, :] = ssq[...]


def _mlp(h, agg, eps, w1, b1, w2, b2):
    return pl.pallas_call(
        _mlp_body,
        grid=(NB,),
        in_specs=[
            pl.BlockSpec((RB, H), lambda i: (i, 0)),
            pl.BlockSpec((RB, H), lambda i: (i, 0)),
            pl.BlockSpec((1, 1), lambda i: (0, 0)),
            pl.BlockSpec((H, H), lambda i: (0, 0)),
            pl.BlockSpec((1, H), lambda i: (0, 0)),
            pl.BlockSpec((H, H), lambda i: (0, 0)),
            pl.BlockSpec((1, H), lambda i: (0, 0)),
        ],
        out_specs=[
            pl.BlockSpec((RB, H), lambda i: (i, 0)),
            pl.BlockSpec((2, H), lambda i: (0, 0)),
        ],
        out_shape=[
            jax.ShapeDtypeStruct((NPAD, H), jnp.float32),
            jax.ShapeDtypeStruct((2, H), jnp.float32),
        ],
        scratch_shapes=[
            pltpu.VMEM((1, H), jnp.float32),
            pltpu.VMEM((1, H), jnp.float32),
        ],
    )(h, agg, eps, w1, b1, w2, b2)


def _bn_body(tr, sr, gr, br, orf):
    mean = sr[0:1, :] * (1.0 / N)
    var = sr[1:2, :] * (1.0 / N) - mean * mean
    inv = lax.rsqrt(var + 1e-5)
    scale = gr[...] * inv
    shift = br[...] - mean * scale
    orf[...] = jnp.maximum(tr[...] * scale + shift, 0.0)


def _bn(t, stats, gamma, beta):
    return pl.pallas_call(
        _bn_body,
        grid=(NB,),
        in_specs=[
            pl.BlockSpec((RB, H), lambda i: (i, 0)),
            pl.BlockSpec((2, H), lambda i: (0, 0)),
            pl.BlockSpec((1, H), lambda i: (0, 0)),
            pl.BlockSpec((1, H), lambda i: (0, 0)),
        ],
        out_specs=pl.BlockSpec((RB, H), lambda i: (i, 0)),
        out_shape=jax.ShapeDtypeStruct((NPAD, H), jnp.float32),
    )(t, stats, gamma, beta)


def _readout_body(ps, cs, w1, b1, w2, b2, orf):
    p = jnp.concatenate([ps[0], ps[1]], axis=1)
    cnt = cs[0, :, 0:1]
    pooled = p / jnp.maximum(cnt, 1.0)
    z = jnp.maximum(jnp.dot(pooled, w1[...], preferred_element_type=jnp.float32) + b1[...], 0.0)
    orf[...] = jnp.dot(z, w2[...], preferred_element_type=jnp.float32) + b2[...]


def _readout(psums, pcnts, w1, b1, w2, b2):
    return pl.pallas_call(
        _readout_body,
        out_shape=jax.ShapeDtypeStruct((G, 1), jnp.float32),
    )(psums, pcnts, w1, b1, w2, b2)


def kernel(x, edge_index, batch, params):
    wp = params["node_proj"]["W"]
    bp = params["node_proj"]["b"].reshape(1, H)

    src = edge_index[0]
    dst = edge_index[1]
    srcp = jnp.concatenate([src, jnp.zeros((EPAD - E,), jnp.int32)])
    dstp = jnp.concatenate([dst, jnp.full((EPAD - E,), N, jnp.int32)])
    sidx_r = jnp.stack([2 * srcp, 2 * srcp + 1]).reshape(2, 16, GROUPS, 1, KB, 128)
    didx_r = jnp.broadcast_to(
        dstp.reshape(1, 16, GROUPS, 1, KB, 128), (2, 16, GROUPS, 1, KB, 128)
    )
    icomb = jnp.concatenate([sidx_r, didx_r], axis=3)
    icomb = jnp.concatenate(
        [icomb, jnp.zeros((2, 16, GPAD - GROUPS, 2, KB, 128), jnp.int32)], axis=2
    )
    batch_r = jnp.concatenate(
        [batch, jnp.full((NPAD - N,), G, jnp.int32)]
    ).reshape(16, 25, 128)

    zpool = jnp.zeros((G + 8, HHALF), jnp.float32)
    ones = jnp.ones((128, HHALF), jnp.float32)

    h = _proj(x, wp, bp)
    for blk in params["blocks"]:
        h2 = h.reshape(2 * NPAD, HHALF)
        agg = _agg_call(h2, icomb)
        t, stats = _mlp(
            h, agg, blk["eps"].reshape(1, 1),
            blk["W1"], blk["b1"].reshape(1, H),
            blk["W2"], blk["b2"].reshape(1, H),
        )
        h = _bn(t, stats, blk["gamma"].reshape(1, H), blk["beta"].reshape(1, H))

    psums, pcnts = _pool_call(h, batch_r, zpool, ones)
    r = params["readout"]
    return _readout(
        psums, pcnts, r["W1"], r["b1"].reshape(1, H), r["W2"], r["b2"].reshape(1, 1)
    )
